# baseline (speedup 1.0000x reference)
import functools

import jax
import jax.numpy as jnp
from jax import lax
from jax.experimental import pallas as pl
from jax.experimental.pallas import tpu as pltpu
from jax.experimental.pallas import tpu_sc as plsc

_B = 16384
_D = 64
_NC = 2
_NS = 16
_NW = _NC * _NS
_BW = _B // _NW
_L = 16


def _make_kernel():
    mesh = plsc.VectorSubcoreMesh(core_axis_name="c", subcore_axis_name="s")

    @functools.partial(
        pl.kernel,
        mesh=mesh,
        compiler_params=pltpu.CompilerParams(
            needs_layout_passes=False, use_tc_tiling_on_sc=False),
        out_type=[
            jax.ShapeDtypeStruct((_B,), jnp.float32),
            jax.ShapeDtypeStruct((_B,), jnp.float32),
        ],
        scratch_types=[
            pltpu.VMEM((_BW,), jnp.int32),
            pltpu.VMEM((_BW,), jnp.int32),
            pltpu.VMEM((_BW,), jnp.int32),
            pltpu.VMEM((_BW, _D), jnp.float32),
            pltpu.VMEM((_BW, _D), jnp.float32),
            pltpu.VMEM((_BW, _D), jnp.float32),
            pltpu.VMEM((_BW,), jnp.float32),
            pltpu.SemaphoreType.DMA,
            pltpu.SemaphoreType.DMA,
            pltpu.SemaphoreType.DMA,
        ],
    )
    def trans_e(ph, pr, pt, nh, nr, nt, ent, rel, pos_out, neg_out,
                idx_h, idx_r, idx_t, hrows, rrows, trows, obuf,
                sem_h, sem_r, sem_t):
        wid = lax.axis_index("s") * _NC + lax.axis_index("c")
        base = wid * _BW
        lanes = lax.iota(jnp.int32, _L)

        def one_side(h_hbm, r_hbm, t_hbm, out_hbm):
            pltpu.sync_copy(h_hbm.at[pl.ds(base, _BW)], idx_h)
            pltpu.sync_copy(r_hbm.at[pl.ds(base, _BW)], idx_r)
            pltpu.sync_copy(t_hbm.at[pl.ds(base, _BW)], idx_t)
            ch = pltpu.async_copy(ent.at[idx_h], hrows, sem_h)
            cr = pltpu.async_copy(rel.at[idx_r], rrows, sem_r)
            ct = pltpu.async_copy(ent.at[idx_t], trows, sem_t)
            ch.wait()
            cr.wait()
            ct.wait()

            def group(g, carry):
                vec = jnp.zeros((_L,), jnp.float32)
                for j in range(_L):
                    i = g * _L + j
                    acc = jnp.zeros((_L,), jnp.float32)
                    for c in range(_D // _L):
                        hv = hrows[i, pl.ds(c * _L, _L)]
                        rv = rrows[i, pl.ds(c * _L, _L)]
                        tv = trows[i, pl.ds(c * _L, _L)]
                        acc = acc + jnp.abs(hv + rv - tv)
                    vec = jnp.where(lanes == j, jnp.sum(acc), vec)
                obuf[pl.ds(g * _L, _L)] = vec
                return carry

            lax.fori_loop(0, _BW // _L, group, 0)
            pltpu.sync_copy(obuf, out_hbm.at[pl.ds(base, _BW)])

        one_side(ph, pr, pt, pos_out)
        one_side(nh, nr, nt, neg_out)

    return trans_e


_KERNEL = _make_kernel()


@jax.jit
def kernel(pos_triples, neg_triples, ent_embs, rel_embs):
    pos = pos_triples.astype(jnp.int32)
    neg = neg_triples.astype(jnp.int32)
    ph, pr, pt = pos[:, 0], pos[:, 1], pos[:, 2]
    nh, nr, nt = neg[:, 0], neg[:, 1], neg[:, 2]
    pos_dist, neg_dist = _KERNEL(ph, pr, pt, nh, nr, nt, ent_embs, rel_embs)
    return pos_dist, neg_dist


# speedup vs baseline: 1.1497x; 1.1497x over previous
import functools

import jax
import jax.numpy as jnp
from jax import lax
from jax.experimental import pallas as pl
from jax.experimental.pallas import tpu as pltpu
from jax.experimental.pallas import tpu_sc as plsc

_B = 16384
_D = 64
_NC = 2
_NS = 16
_NW = _NC * _NS
_BW = _B // _NW
_L = 16


def _make_kernel():
    mesh = plsc.VectorSubcoreMesh(core_axis_name="c", subcore_axis_name="s")

    @functools.partial(
        pl.kernel,
        mesh=mesh,
        compiler_params=pltpu.CompilerParams(
            needs_layout_passes=False, use_tc_tiling_on_sc=True),
        out_type=[
            jax.ShapeDtypeStruct((_B,), jnp.float32),
            jax.ShapeDtypeStruct((_B,), jnp.float32),
        ],
        scratch_types=[
            pltpu.VMEM((_BW,), jnp.int32),
            pltpu.VMEM((_BW,), jnp.int32),
            pltpu.VMEM((_BW,), jnp.int32),
            pltpu.VMEM((_L, 8, _D), jnp.float32),
            pltpu.VMEM((_L, 8, _D), jnp.float32),
            pltpu.VMEM((_L, 8, _D), jnp.float32),
            pltpu.VMEM((_BW,), jnp.float32),
            pltpu.SemaphoreType.DMA,
            pltpu.SemaphoreType.DMA,
            pltpu.SemaphoreType.DMA,
        ],
    )
    def trans_e(ph, pr, pt, nh, nr, nt, ent, rel, pos_out, neg_out,
                idx_h, idx_r, idx_t, hwin, rwin, twin, obuf,
                sem_h, sem_r, sem_t):
        wid = lax.axis_index("s") * _NC + lax.axis_index("c")
        base = wid * _BW
        lanes = lax.iota(jnp.int32, _L)

        def one_side(h_hbm, r_hbm, t_hbm, out_hbm):
            pltpu.sync_copy(h_hbm.at[pl.ds(base, _BW)], idx_h)
            pltpu.sync_copy(r_hbm.at[pl.ds(base, _BW)], idx_r)
            pltpu.sync_copy(t_hbm.at[pl.ds(base, _BW)], idx_t)

            def group(g, carry):
                iv_h = idx_h[pl.ds(g * _L, _L)]
                iv_r = idx_r[pl.ds(g * _L, _L)]
                iv_t = idx_t[pl.ds(g * _L, _L)]
                bh = lax.shift_left(lax.shift_right_logical(iv_h, 3), 3)
                br = lax.shift_left(lax.shift_right_logical(iv_r, 3), 3)
                bt = lax.shift_left(lax.shift_right_logical(iv_t, 3), 3)
                sh = lax.bitwise_and(iv_h, 7)
                sr = lax.bitwise_and(iv_r, 7)
                st = lax.bitwise_and(iv_t, 7)
                for j in range(_L):
                    pltpu.async_copy(
                        ent.at[pl.ds(pl.multiple_of(bh[j], 8), 8)],
                        hwin.at[j], sem_h)
                    pltpu.async_copy(
                        rel.at[pl.ds(pl.multiple_of(br[j], 8), 8)],
                        rwin.at[j], sem_r)
                    pltpu.async_copy(
                        ent.at[pl.ds(pl.multiple_of(bt[j], 8), 8)],
                        twin.at[j], sem_t)
                for j in range(_L):
                    pltpu.make_async_copy(
                        ent.at[pl.ds(0, 8)], hwin.at[j], sem_h).wait()
                    pltpu.make_async_copy(
                        rel.at[pl.ds(0, 8)], rwin.at[j], sem_r).wait()
                    pltpu.make_async_copy(
                        ent.at[pl.ds(0, 8)], twin.at[j], sem_t).wait()
                vec = jnp.zeros((_L,), jnp.float32)
                for j in range(_L):
                    acc = jnp.zeros((_L,), jnp.float32)
                    for c in range(_D // _L):
                        hv = hwin[j, sh[j], pl.ds(c * _L, _L)]
                        rv = rwin[j, sr[j], pl.ds(c * _L, _L)]
                        tv = twin[j, st[j], pl.ds(c * _L, _L)]
                        acc = acc + jnp.abs(hv + rv - tv)
                    vec = jnp.where(lanes == j, jnp.sum(acc), vec)
                obuf[pl.ds(g * _L, _L)] = vec
                return carry

            lax.fori_loop(0, _BW // _L, group, 0)
            pltpu.sync_copy(obuf, out_hbm.at[pl.ds(base, _BW)])

        one_side(ph, pr, pt, pos_out)
        one_side(nh, nr, nt, neg_out)

    return trans_e


_KERNEL = _make_kernel()


@jax.jit
def kernel(pos_triples, neg_triples, ent_embs, rel_embs):
    pos = pos_triples.astype(jnp.int32)
    neg = neg_triples.astype(jnp.int32)
    ph, pr, pt = pos[:, 0], pos[:, 1], pos[:, 2]
    nh, nr, nt = neg[:, 0], neg[:, 1], neg[:, 2]
    pos_dist, neg_dist = _KERNEL(ph, pr, pt, nh, nr, nt, ent_embs, rel_embs)
    return pos_dist, neg_dist


# speedup vs baseline: 1.2173x; 1.0587x over previous
import functools

import jax
import jax.numpy as jnp
from jax import lax
from jax.experimental import pallas as pl
from jax.experimental.pallas import tpu as pltpu
from jax.experimental.pallas import tpu_sc as plsc

_B = 16384
_D = 64
_NC = 2
_NS = 16
_NW = _NC * _NS
_BW = _B // _NW
_L = 16
_NG = _BW // _L


def _make_kernel():
    mesh = plsc.VectorSubcoreMesh(core_axis_name="c", subcore_axis_name="s")

    win = lambda: pltpu.VMEM((_L, 8, _D), jnp.float32)

    @functools.partial(
        pl.kernel,
        mesh=mesh,
        compiler_params=pltpu.CompilerParams(
            needs_layout_passes=False, use_tc_tiling_on_sc=True),
        out_type=[
            jax.ShapeDtypeStruct((_B,), jnp.float32),
            jax.ShapeDtypeStruct((_B,), jnp.float32),
        ],
        scratch_types=[
            pltpu.VMEM((_BW,), jnp.int32),
            pltpu.VMEM((_BW,), jnp.int32),
            pltpu.VMEM((_BW,), jnp.int32),
            win(), win(), win(),
            win(), win(), win(),
            pltpu.VMEM((_BW,), jnp.float32),
            pltpu.SemaphoreType.DMA,
            pltpu.SemaphoreType.DMA,
            pltpu.SemaphoreType.DMA,
            pltpu.SemaphoreType.DMA,
            pltpu.SemaphoreType.DMA,
            pltpu.SemaphoreType.DMA,
        ],
    )
    def trans_e(ph, pr, pt, nh, nr, nt, ent, rel, pos_out, neg_out,
                idx_h, idx_r, idx_t,
                ha, ra, ta, hb, rb, tb, obuf,
                sha, sra, sta, shb, srb, stb):
        wid = lax.axis_index("s") * _NC + lax.axis_index("c")
        base = wid * _BW
        lanes = lax.iota(jnp.int32, _L)
        bufs = ((ha, ra, ta, sha, sra, sta),
                (hb, rb, tb, shb, srb, stb))

        def issue(g, buf):
            hw, rw, tw, s1, s2, s3 = buf
            iv_h = idx_h[pl.ds(g * _L, _L)]
            iv_r = idx_r[pl.ds(g * _L, _L)]
            iv_t = idx_t[pl.ds(g * _L, _L)]
            bh = lax.shift_left(lax.shift_right_logical(iv_h, 3), 3)
            br = lax.shift_left(lax.shift_right_logical(iv_r, 3), 3)
            bt = lax.shift_left(lax.shift_right_logical(iv_t, 3), 3)
            for j in range(_L):
                pltpu.async_copy(
                    ent.at[pl.ds(pl.multiple_of(bh[j], 8), 8)],
                    hw.at[j], s1)
                pltpu.async_copy(
                    rel.at[pl.ds(pl.multiple_of(br[j], 8), 8)],
                    rw.at[j], s2)
                pltpu.async_copy(
                    ent.at[pl.ds(pl.multiple_of(bt[j], 8), 8)],
                    tw.at[j], s3)

        def drain(buf):
            hw, rw, tw, s1, s2, s3 = buf
            for j in range(_L):
                pltpu.make_async_copy(
                    ent.at[pl.ds(0, 8)], hw.at[j], s1).wait()
                pltpu.make_async_copy(
                    rel.at[pl.ds(0, 8)], rw.at[j], s2).wait()
                pltpu.make_async_copy(
                    ent.at[pl.ds(0, 8)], tw.at[j], s3).wait()

        def compute(g, buf):
            hw, rw, tw, _, _, _ = buf
            iv_h = idx_h[pl.ds(g * _L, _L)]
            iv_r = idx_r[pl.ds(g * _L, _L)]
            iv_t = idx_t[pl.ds(g * _L, _L)]
            sh = lax.bitwise_and(iv_h, 7)
            sr = lax.bitwise_and(iv_r, 7)
            st = lax.bitwise_and(iv_t, 7)
            vec = jnp.zeros((_L,), jnp.float32)
            for j in range(_L):
                acc = jnp.zeros((_L,), jnp.float32)
                for c in range(_D // _L):
                    hv = hw[j, sh[j], pl.ds(c * _L, _L)]
                    rv = rw[j, sr[j], pl.ds(c * _L, _L)]
                    tv = tw[j, st[j], pl.ds(c * _L, _L)]
                    acc = acc + jnp.abs(hv + rv - tv)
                vec = jnp.where(lanes == j, jnp.sum(acc), vec)
            obuf[pl.ds(g * _L, _L)] = vec

        def one_side(h_hbm, r_hbm, t_hbm, out_hbm):
            pltpu.sync_copy(h_hbm.at[pl.ds(base, _BW)], idx_h)
            pltpu.sync_copy(r_hbm.at[pl.ds(base, _BW)], idx_r)
            pltpu.sync_copy(t_hbm.at[pl.ds(base, _BW)], idx_t)
            issue(0, bufs[0])

            def body(k, carry):
                g_a = 2 * k
                g_b = 2 * k + 1
                issue(g_b, bufs[1])
                drain(bufs[0])
                compute(g_a, bufs[0])
                issue(jnp.minimum(g_b + 1, _NG - 1), bufs[0])
                drain(bufs[1])
                compute(g_b, bufs[1])
                return carry

            lax.fori_loop(0, _NG // 2, body, 0)
            drain(bufs[0])
            pltpu.sync_copy(obuf, out_hbm.at[pl.ds(base, _BW)])

        one_side(ph, pr, pt, pos_out)
        one_side(nh, nr, nt, neg_out)

    return trans_e


_KERNEL = _make_kernel()


@jax.jit
def kernel(pos_triples, neg_triples, ent_embs, rel_embs):
    pos = pos_triples.astype(jnp.int32)
    neg = neg_triples.astype(jnp.int32)
    ph, pr, pt = pos[:, 0], pos[:, 1], pos[:, 2]
    nh, nr, nt = neg[:, 0], neg[:, 1], neg[:, 2]
    pos_dist, neg_dist = _KERNEL(ph, pr, pt, nh, nr, nt, ent_embs, rel_embs)
    return pos_dist, neg_dist


# speedup vs baseline: 1.2219x; 1.0038x over previous
import functools

import jax
import jax.numpy as jnp
from jax import lax
from jax.experimental import pallas as pl
from jax.experimental.pallas import tpu as pltpu
from jax.experimental.pallas import tpu_sc as plsc

_B = 16384
_D = 64
_NC = 2
_NS = 16
_NW = _NC * _NS
_BW = _B // _NW
_L = 16
_NG = _BW // _L


def _make_kernel():
    mesh = plsc.VectorSubcoreMesh(core_axis_name="c", subcore_axis_name="s")

    win = lambda: pltpu.VMEM((_L * 8, _D), jnp.float32)

    @functools.partial(
        pl.kernel,
        mesh=mesh,
        compiler_params=pltpu.CompilerParams(
            needs_layout_passes=False, use_tc_tiling_on_sc=True),
        out_type=[
            jax.ShapeDtypeStruct((_B,), jnp.float32),
            jax.ShapeDtypeStruct((_B,), jnp.float32),
        ],
        scratch_types=[
            pltpu.VMEM((_BW,), jnp.int32),
            pltpu.VMEM((_BW,), jnp.int32),
            pltpu.VMEM((_BW,), jnp.int32),
            win(), win(), win(),
            win(), win(), win(),
            pltpu.VMEM((_BW,), jnp.float32),
            pltpu.SemaphoreType.DMA,
            pltpu.SemaphoreType.DMA,
            pltpu.SemaphoreType.DMA,
            pltpu.SemaphoreType.DMA,
            pltpu.SemaphoreType.DMA,
            pltpu.SemaphoreType.DMA,
        ],
    )
    def trans_e(ph, pr, pt, nh, nr, nt, ent, rel, pos_out, neg_out,
                idx_h, idx_r, idx_t,
                ha, ra, ta, hb, rb, tb, obuf,
                sha, sra, sta, shb, srb, stb):
        wid = lax.axis_index("s") * _NC + lax.axis_index("c")
        base = wid * _BW
        lanes = lax.iota(jnp.int32, _L)
        bufs = ((ha, ra, ta, sha, sra, sta),
                (hb, rb, tb, shb, srb, stb))

        def issue(g, buf):
            hw, rw, tw, s1, s2, s3 = buf
            iv_h = idx_h[pl.ds(g * _L, _L)]
            iv_r = idx_r[pl.ds(g * _L, _L)]
            iv_t = idx_t[pl.ds(g * _L, _L)]
            bh = lax.shift_left(lax.shift_right_logical(iv_h, 3), 3)
            br = lax.shift_left(lax.shift_right_logical(iv_r, 3), 3)
            bt = lax.shift_left(lax.shift_right_logical(iv_t, 3), 3)
            for j in range(_L):
                pltpu.async_copy(
                    ent.at[pl.ds(pl.multiple_of(bh[j], 8), 8)],
                    hw.at[pl.ds(8 * j, 8)], s1)
                pltpu.async_copy(
                    rel.at[pl.ds(pl.multiple_of(br[j], 8), 8)],
                    rw.at[pl.ds(8 * j, 8)], s2)
                pltpu.async_copy(
                    ent.at[pl.ds(pl.multiple_of(bt[j], 8), 8)],
                    tw.at[pl.ds(8 * j, 8)], s3)

        def drain(buf):
            hw, rw, tw, s1, s2, s3 = buf
            pltpu.make_async_copy(
                ent.at[pl.ds(0, _L * 8)], hw, s1).wait()
            pltpu.make_async_copy(
                ent.at[pl.ds(0, _L * 8)], rw, s2).wait()
            pltpu.make_async_copy(
                ent.at[pl.ds(0, _L * 8)], tw, s3).wait()

        def compute(g, buf):
            hw, rw, tw, _, _, _ = buf
            iv_h = idx_h[pl.ds(g * _L, _L)]
            iv_r = idx_r[pl.ds(g * _L, _L)]
            iv_t = idx_t[pl.ds(g * _L, _L)]
            sh = lax.bitwise_and(iv_h, 7)
            sr = lax.bitwise_and(iv_r, 7)
            st = lax.bitwise_and(iv_t, 7)
            vec = jnp.zeros((_L,), jnp.float32)
            for j in range(_L):
                acc = jnp.zeros((_L,), jnp.float32)
                for c in range(_D // _L):
                    hv = hw[8 * j + sh[j], pl.ds(c * _L, _L)]
                    rv = rw[8 * j + sr[j], pl.ds(c * _L, _L)]
                    tv = tw[8 * j + st[j], pl.ds(c * _L, _L)]
                    acc = acc + jnp.abs(hv + rv - tv)
                vec = jnp.where(lanes == j, jnp.sum(acc), vec)
            obuf[pl.ds(g * _L, _L)] = vec

        def one_side(h_hbm, r_hbm, t_hbm, out_hbm):
            pltpu.sync_copy(h_hbm.at[pl.ds(base, _BW)], idx_h)
            pltpu.sync_copy(r_hbm.at[pl.ds(base, _BW)], idx_r)
            pltpu.sync_copy(t_hbm.at[pl.ds(base, _BW)], idx_t)
            issue(0, bufs[0])

            def body(k, carry):
                g_a = 2 * k
                g_b = 2 * k + 1
                issue(g_b, bufs[1])
                drain(bufs[0])
                compute(g_a, bufs[0])
                issue(jnp.minimum(g_b + 1, _NG - 1), bufs[0])
                drain(bufs[1])
                compute(g_b, bufs[1])
                return carry

            lax.fori_loop(0, _NG // 2, body, 0)
            drain(bufs[0])
            pltpu.sync_copy(obuf, out_hbm.at[pl.ds(base, _BW)])

        one_side(ph, pr, pt, pos_out)
        one_side(nh, nr, nt, neg_out)

    return trans_e


_KERNEL = _make_kernel()


@jax.jit
def kernel(pos_triples, neg_triples, ent_embs, rel_embs):
    pos = pos_triples.astype(jnp.int32)
    neg = neg_triples.astype(jnp.int32)
    ph, pr, pt = pos[:, 0], pos[:, 1], pos[:, 2]
    nh, nr, nt = neg[:, 0], neg[:, 1], neg[:, 2]
    pos_dist, neg_dist = _KERNEL(ph, pr, pt, nh, nr, nt, ent_embs, rel_embs)
    return pos_dist, neg_dist
